# baseline (speedup 1.0000x reference)
import jax
import jax.numpy as jnp
from jax.experimental import pallas as pl


def _leaky_relu(x, alpha=0.2):
    return jnp.where(x >= 0, x, alpha * x)


def _elu(x):
    return jnp.where(x > 0, x, jnp.exp(jnp.minimum(x, 0.0)) - 1.0)


def _gat_block(feats, asn, neg_mask, H, C):
    N = feats.shape[0]
    acc = jnp.zeros((N, C), jnp.float32)
    inv_h = 1.0 / H
    feats_b = feats.astype(jnp.bfloat16)
    esen = jnp.dot(feats, asn, preferred_element_type=jnp.float32)
    enT = jnp.transpose(esen[:, H:])
    for h in range(H):
        t = esen[:, h:h + 1] + enT[h:h + 1, :]
        p = jnp.exp2(jnp.maximum(t, 0.2 * t) + neg_mask)
        pb = p.astype(jnp.bfloat16)
        s = jnp.sum(p, axis=1, keepdims=True)
        acc = acc + jnp.dot(pb, feats_b[:, h * C:(h + 1) * C],
                            preferred_element_type=jnp.float32) * (inv_h / s)
    return acc


def _body(x_ref, idx_ref, a_ref, down_ref, orig_ref,
          wup_ref, asn_up_ref, wend_ref, asn_end_ref, out_ref):
    No, F = x_ref.shape[1], x_ref.shape[2]
    Nn = a_ref.shape[1]
    H = asn_up_ref.shape[1] // 2
    C = wup_ref.shape[1] // H
    x = x_ref[0]
    idx = idx_ref[0, 0, :]

    rows = jax.lax.broadcasted_iota(jnp.int32, (Nn, No), 0)
    onehot = (rows == idx[None, :]).astype(jnp.float32)
    xu = jnp.dot(onehot, x, preferred_element_type=jnp.float32)

    a = a_ref[0]
    ri = jax.lax.broadcasted_iota(jnp.int32, (Nn, Nn), 0)
    ci = jax.lax.broadcasted_iota(jnp.int32, (Nn, Nn), 1)
    edge = jnp.logical_or(a > 0.5, ri == ci)
    neg_mask = jnp.where(edge, 0.0, -1e9).astype(jnp.float32)

    feats1 = jnp.dot(xu, wup_ref[...], preferred_element_type=jnp.float32)
    x1 = _elu(_gat_block(feats1, asn_up_ref[...], neg_mask, H, C))
    x1 = x1 + down_ref[0]

    feats2 = (jnp.dot(x1, wend_ref[:F, :], preferred_element_type=jnp.float32)
              + jnp.dot(orig_ref[0], wend_ref[F:, :],
                        preferred_element_type=jnp.float32))
    out_ref[0] = _elu(_gat_block(feats2, asn_end_ref[...], neg_mask, H, C))


def kernel(X, orig_X, l_n, idx0, A0, down0, action, W_up, a_s_up, a_n_up,
           W_end, a_s_end, a_n_end):
    B, No, F = X.shape
    Nn = A0.shape[1]
    H, C = a_s_up.shape
    idx3 = idx0.astype(jnp.int32).reshape(B, 1, No)
    wup = W_up.reshape(F, H * C)
    wend = W_end.reshape(2 * F, H * C)

    log2e = 1.4426950408889634
    eye = jnp.eye(H, dtype=jnp.float32)
    asn_up = jnp.concatenate(
        [(a_s_up[:, :, None] * eye[:, None, :]).reshape(H * C, H),
         (a_n_up[:, :, None] * eye[:, None, :]).reshape(H * C, H)],
        axis=1) * log2e
    asn_end = jnp.concatenate(
        [(a_s_end[:, :, None] * eye[:, None, :]).reshape(H * C, H),
         (a_n_end[:, :, None] * eye[:, None, :]).reshape(H * C, H)],
        axis=1) * log2e

    full = lambda *shape: pl.BlockSpec(shape, lambda b: (0,) * len(shape))
    out = pl.pallas_call(
        _body,
        grid=(B,),
        in_specs=[
            pl.BlockSpec((1, No, F), lambda b: (b, 0, 0)),
            pl.BlockSpec((1, 1, No), lambda b: (b, 0, 0)),
            pl.BlockSpec((1, Nn, Nn), lambda b: (b, 0, 0)),
            pl.BlockSpec((1, Nn, F), lambda b: (b, 0, 0)),
            pl.BlockSpec((1, Nn, F), lambda b: (b, 0, 0)),
            full(F, H * C),
            full(H * C, 2 * H),
            full(2 * F, H * C),
            full(H * C, 2 * H),
        ],
        out_specs=pl.BlockSpec((1, Nn, F), lambda b: (b, 0, 0)),
        out_shape=jax.ShapeDtypeStruct((B, Nn, F), jnp.float32),
    )(X, idx3, A0, down0, orig_X, wup, asn_up, wend, asn_end)

    scale = (jnp.asarray(l_n) / 1).astype(out.dtype)
    return out * scale


# speedup vs baseline: 1.0174x; 1.0174x over previous
import jax
import jax.numpy as jnp
from jax.experimental import pallas as pl
from jax.experimental.pallas import tpu as pltpu


def _leaky_relu(x, alpha=0.2):
    return jnp.where(x >= 0, x, alpha * x)


def _elu(x):
    return jnp.where(x > 0, x, jnp.exp(jnp.minimum(x, 0.0)) - 1.0)


def _gat_block(feats, asn, neg_mask, H, C):
    N = feats.shape[0]
    acc = jnp.zeros((N, C), jnp.float32)
    inv_h = 1.0 / H
    feats_b = feats.astype(jnp.bfloat16)
    esen = jnp.dot(feats, asn, preferred_element_type=jnp.float32)
    enT = jnp.transpose(esen[:, H:])
    for h in range(H):
        t = esen[:, h:h + 1] + enT[h:h + 1, :]
        p = jnp.exp2(jnp.maximum(t, 0.2 * t) + neg_mask)
        pb = p.astype(jnp.bfloat16)
        s = jnp.sum(p, axis=1, keepdims=True)
        acc = acc + jnp.dot(pb, feats_b[:, h * C:(h + 1) * C],
                            preferred_element_type=jnp.float32) * (inv_h / s)
    return acc


def _body(scale_ref, x_ref, idx_ref, a_ref, down_ref, orig_ref,
          wup_ref, asn_up_ref, wend_ref, asn_end_ref, out_ref):
    No, F = x_ref.shape[1], x_ref.shape[2]
    Nn = a_ref.shape[1]
    H = asn_up_ref.shape[1] // 2
    C = wup_ref.shape[1] // H
    x = x_ref[0]
    idx = idx_ref[0, 0, :]

    rows = jax.lax.broadcasted_iota(jnp.int32, (Nn, No), 0)
    onehot = (rows == idx[None, :]).astype(jnp.float32)
    xu = jnp.dot(onehot, x, preferred_element_type=jnp.float32)

    a = a_ref[0]
    ri = jax.lax.broadcasted_iota(jnp.int32, (Nn, Nn), 0)
    ci = jax.lax.broadcasted_iota(jnp.int32, (Nn, Nn), 1)
    edge = jnp.logical_or(a > 0.5, ri == ci)
    neg_mask = jnp.where(edge, 0.0, -1e9).astype(jnp.float32)

    feats1 = jnp.dot(xu, wup_ref[...], preferred_element_type=jnp.float32)
    x1 = _elu(_gat_block(feats1, asn_up_ref[...], neg_mask, H, C))
    x1 = x1 + down_ref[0]

    feats2 = (jnp.dot(x1, wend_ref[:F, :], preferred_element_type=jnp.float32)
              + jnp.dot(orig_ref[0], wend_ref[F:, :],
                        preferred_element_type=jnp.float32))
    out_ref[0] = _elu(_gat_block(feats2, asn_end_ref[...], neg_mask,
                                 H, C)) * scale_ref[0]


def kernel(X, orig_X, l_n, idx0, A0, down0, action, W_up, a_s_up, a_n_up,
           W_end, a_s_end, a_n_end):
    B, No, F = X.shape
    Nn = A0.shape[1]
    H, C = a_s_up.shape
    idx3 = idx0.astype(jnp.int32).reshape(B, 1, No)
    wup = W_up.reshape(F, H * C)
    wend = W_end.reshape(2 * F, H * C)

    log2e = 1.4426950408889634
    eye = jnp.eye(H, dtype=jnp.float32)
    asn_up = jnp.concatenate(
        [(a_s_up[:, :, None] * eye[:, None, :]).reshape(H * C, H),
         (a_n_up[:, :, None] * eye[:, None, :]).reshape(H * C, H)],
        axis=1) * log2e
    asn_end = jnp.concatenate(
        [(a_s_end[:, :, None] * eye[:, None, :]).reshape(H * C, H),
         (a_n_end[:, :, None] * eye[:, None, :]).reshape(H * C, H)],
        axis=1) * log2e

    scale = (jnp.asarray(l_n) / 1).astype(jnp.float32).reshape(1)

    full = lambda *shape: pl.BlockSpec(shape, lambda b: (0,) * len(shape))
    out = pl.pallas_call(
        _body,
        grid=(B,),
        in_specs=[
            pl.BlockSpec(memory_space=pltpu.SMEM),
            pl.BlockSpec((1, No, F), lambda b: (b, 0, 0)),
            pl.BlockSpec((1, 1, No), lambda b: (b, 0, 0)),
            pl.BlockSpec((1, Nn, Nn), lambda b: (b, 0, 0)),
            pl.BlockSpec((1, Nn, F), lambda b: (b, 0, 0)),
            pl.BlockSpec((1, Nn, F), lambda b: (b, 0, 0)),
            full(F, H * C),
            full(H * C, 2 * H),
            full(2 * F, H * C),
            full(H * C, 2 * H),
        ],
        out_specs=pl.BlockSpec((1, Nn, F), lambda b: (b, 0, 0)),
        out_shape=jax.ShapeDtypeStruct((B, Nn, F), jnp.float32),
    )(scale, X, idx3, A0, down0, orig_X, wup, asn_up, wend, asn_end)
    return out
